# baseline (speedup 1.0000x reference)
import functools

import jax
import jax.numpy as jnp
from jax import lax
from jax.experimental import pallas as pl
from jax.experimental.pallas import tpu as pltpu
from jax.experimental.pallas import tpu_sc as plsc

NUM_USERS = 1000000
NUM_ITEMS = 1000000
EMBED_DIM = 64
BATCH = 16384

_info = plsc.get_sparse_core_info()
NC = _info.num_cores
NS = _info.num_subcores
L = _info.num_lanes
NW = NC * NS
BPW = BATCH // NW
GROUPS = BPW // L


def _make_kernel():
    mesh = plsc.VectorSubcoreMesh(core_axis_name="c", subcore_axis_name="s")

    @functools.partial(
        pl.kernel,
        mesh=mesh,
        out_type=jax.ShapeDtypeStruct((BATCH,), jnp.float32),
        compiler_params=pltpu.CompilerParams(use_tc_tiling_on_sc=False),
        scratch_types=[
            pltpu.VMEM((BPW,), jnp.int32),
            pltpu.VMEM((BPW,), jnp.int32),
            pltpu.VMEM((BPW, EMBED_DIM), jnp.float32),
            pltpu.VMEM((BPW, EMBED_DIM), jnp.float32),
            pltpu.VMEM((BPW,), jnp.float32),
            pltpu.SemaphoreType.DMA,
            pltpu.SemaphoreType.DMA,
        ],
    )
    def k(user_hbm, item_hbm, uid_hbm, iid_hbm, out_hbm,
          uid_v, iid_v, urows_v, irows_v, out_v, sem_u, sem_i):
        wid = lax.axis_index("s") * NC + lax.axis_index("c")
        base = wid * BPW

        pltpu.sync_copy(uid_hbm.at[pl.ds(base, BPW)], uid_v)
        pltpu.sync_copy(iid_hbm.at[pl.ds(base, BPW)], iid_v)

        cp_u = pltpu.async_copy(user_hbm.at[uid_v], urows_v, sem_u)
        cp_i = pltpu.async_copy(item_hbm.at[iid_v], irows_v, sem_i)
        cp_u.wait()
        cp_i.wait()

        lane = lax.iota(jnp.int32, 16)
        perms = [lane ^ m for m in (1, 2, 4, 8)]

        dnums = lax.GatherDimensionNumbers(
            offset_dims=(), collapsed_slice_dims=(0,), start_index_map=(0,))

        def shuffle(x, idx):
            return lax.gather(x, idx[:, None], dnums, (1,),
                              mode=lax.GatherScatterMode.PROMISE_IN_BOUNDS)

        def lanesum(s):
            for p in perms:
                s = s + shuffle(s, p)
            return s

        def group(g, carry):
            res = jnp.zeros((L,), jnp.float32)
            for r in range(L):
                row = g * L + r
                s = urows_v[row, pl.ds(0, L)] * irows_v[row, pl.ds(0, L)]
                for c in range(1, EMBED_DIM // L):
                    s = s + (urows_v[row, pl.ds(c * L, L)]
                             * irows_v[row, pl.ds(c * L, L)])
                res = jnp.where(lane == r, lanesum(s), res)
            y = 1.0 / (1.0 + jnp.exp(-res))
            out_v[pl.ds(g * L, L)] = y
            return carry

        lax.fori_loop(0, GROUPS, group, 0)

        pltpu.sync_copy(out_v, out_hbm.at[pl.ds(base, BPW)])

    return k


_kernel_call = _make_kernel()


def kernel(user_emb, item_emb, user_id, item_id):
    uid = jnp.asarray(user_id, jnp.int32)
    iid = jnp.asarray(item_id, jnp.int32)
    return _kernel_call(user_emb, item_emb, uid, iid)


# speedup vs baseline: 2.6056x; 2.6056x over previous
import functools

import jax
import jax.numpy as jnp
from jax import lax
from jax.experimental import pallas as pl
from jax.experimental.pallas import tpu as pltpu
from jax.experimental.pallas import tpu_sc as plsc

NUM_USERS = 1000000
NUM_ITEMS = 1000000
EMBED_DIM = 64
BATCH = 16384

_info = plsc.get_sparse_core_info()
NC = _info.num_cores
NS = _info.num_subcores
L = _info.num_lanes
NW = NC * NS
BPW = BATCH // NW
BLOCKS = BPW // L
NSLOTS = 4


def _make_kernel():
    mesh = plsc.VectorSubcoreMesh(core_axis_name="c", subcore_axis_name="s")

    @functools.partial(
        pl.kernel,
        mesh=mesh,
        out_type=jax.ShapeDtypeStruct((BATCH,), jnp.float32),
        compiler_params=pltpu.CompilerParams(needs_layout_passes=False),
        scratch_types=[
            pltpu.VMEM((BPW,), jnp.int32),
            pltpu.VMEM((BPW,), jnp.int32),
            pltpu.VMEM((NSLOTS, EMBED_DIM, 128), jnp.float32),
            pltpu.VMEM((NSLOTS, EMBED_DIM, 128), jnp.float32),
            pltpu.VMEM((BPW,), jnp.float32),
            [pltpu.SemaphoreType.DMA] * NSLOTS,
            [pltpu.SemaphoreType.DMA] * NSLOTS,
        ],
    )
    def k(ut_hbm, it_hbm, uid_hbm, iid_hbm, out_hbm,
          uid_v, iid_v, upan_v, ipan_v, out_v, sems_u, sems_i):
        wid = lax.axis_index("s") * NC + lax.axis_index("c")
        base = wid * BPW

        pltpu.sync_copy(uid_hbm.at[pl.ds(base, BPW)], uid_v)
        pltpu.sync_copy(iid_hbm.at[pl.ds(base, BPW)], iid_v)

        lane = lax.iota(jnp.int32, 16)
        perms = [lane ^ m for m in (1, 2, 4, 8)]
        rowidx = [lane + 16 * kk for kk in range(EMBED_DIM // L)]
        jconst = [jnp.full((L,), j, jnp.int32) for j in range(L)]

        dnums = lax.GatherDimensionNumbers(
            offset_dims=(), collapsed_slice_dims=(0,), start_index_map=(0,))

        def shuffle(x, idx):
            return lax.gather(x, idx[:, None], dnums, (1,),
                              mode=lax.GatherScatterMode.PROMISE_IN_BOUNDS)

        def lanesum(s):
            for p in perms:
                s = s + shuffle(s, p)
            return s

        def issue(j, pu, pi):
            slot = j % NSLOTS
            pb_u = pl.multiple_of((pu[j] >> 7) << 7, 128)
            pb_i = pl.multiple_of((pi[j] >> 7) << 7, 128)
            pltpu.async_copy(ut_hbm.at[:, pl.ds(pb_u, 128)],
                             upan_v.at[slot], sems_u[slot])
            pltpu.async_copy(it_hbm.at[:, pl.ds(pb_i, 128)],
                             ipan_v.at[slot], sems_i[slot])

        def block(blk, carry):
            pu = uid_v[pl.ds(blk * L, L)]
            pi = iid_v[pl.ds(blk * L, L)]
            cu = pu & 127
            ci = pi & 127
            for j in range(NSLOTS):
                issue(j, pu, pi)
            res = jnp.zeros((L,), jnp.float32)
            for j in range(L):
                slot = j % NSLOTS
                pltpu.make_async_copy(ut_hbm.at[:, pl.ds(0, 128)],
                                      upan_v.at[slot], sems_u[slot]).wait()
                pltpu.make_async_copy(it_hbm.at[:, pl.ds(0, 128)],
                                      ipan_v.at[slot], sems_i[slot]).wait()
                cuj = shuffle(cu, jconst[j])
                cij = shuffle(ci, jconst[j])
                s = None
                for kk in range(EMBED_DIM // L):
                    uu = plsc.load_gather(upan_v.at[slot], [rowidx[kk], cuj])
                    ii = plsc.load_gather(ipan_v.at[slot], [rowidx[kk], cij])
                    s = uu * ii if s is None else s + uu * ii
                if j + NSLOTS < L:
                    issue(j + NSLOTS, pu, pi)
                res = jnp.where(lane == j, lanesum(s), res)
            y = 1.0 / (1.0 + jnp.exp(-res))
            out_v[pl.ds(blk * L, L)] = y
            return carry

        lax.fori_loop(0, BLOCKS, block, 0)

        pltpu.sync_copy(out_v, out_hbm.at[pl.ds(base, BPW)])

    return k


_kernel_call = _make_kernel()


def kernel(user_emb, item_emb, user_id, item_id):
    uid = jnp.asarray(user_id, jnp.int32)
    iid = jnp.asarray(item_id, jnp.int32)
    return _kernel_call(user_emb.T, item_emb.T, uid, iid)
